# Initial kernel scaffold; baseline (speedup 1.0000x reference)
import functools

import jax
import jax.numpy as jnp
from jax import lax
from jax.experimental import pallas as pl
from jax.experimental.pallas import tpu as pltpu
from jax.experimental.pallas import tpu_sc as plsc

N = 10000
E = 320000
SD = 16
OC = 16
ED = 16
HID = 16
NPAD = 10240

NC = 2
NS = 16
NW = NC * NS
EPW = E // NW
CH = 2000
NCH = EPW // CH
RPT = NPAD // NS


def _sc_gather(x, src):
    mesh = plsc.VectorSubcoreMesh(core_axis_name="c", subcore_axis_name="s")

    @functools.partial(
        pl.kernel,
        out_type=jax.ShapeDtypeStruct((E, SD), jnp.float32),
        mesh=mesh,
        scratch_types=[
            pltpu.VMEM((CH,), jnp.int32),
            pltpu.VMEM((CH, SD), jnp.float32),
            pltpu.SemaphoreType.DMA,
        ],
    )
    def gather_k(x_hbm, src_hbm, xj_hbm, idx_v, rows_v, sem):
        cid = lax.axis_index("c")
        sid = lax.axis_index("s")
        base = (sid * NC + cid) * EPW

        def body(j, carry):
            off = base + j * CH
            pltpu.sync_copy(src_hbm.at[pl.ds(off, CH)], idx_v)
            pltpu.async_copy(x_hbm.at[idx_v], rows_v, sem).wait()
            pltpu.sync_copy(rows_v, xj_hbm.at[pl.ds(off, CH)])
            return carry

        lax.fori_loop(0, NCH, body, 0)

    return gather_k(x, src)


def _tc_msg(ea, xj, W1, b1, W2, b2):
    TE = 2000
    grid = (E // TE,)

    def body(ea_ref, xj_ref, w1_ref, b1_ref, w2_ref, b2_ref, out_ref):
        h = jnp.maximum(
            jnp.dot(ea_ref[...], w1_ref[...], preferred_element_type=jnp.float32)
            + b1_ref[...], 0.0)
        w = jnp.dot(h, w2_ref[...], preferred_element_type=jnp.float32) + b2_ref[...]
        xjv = xj_ref[...]
        acc = xjv[:, 0:1] * w[:, 0:OC]
        for i in range(1, SD):
            acc = acc + xjv[:, i:i + 1] * w[:, i * OC:(i + 1) * OC]
        out_ref[...] = acc

    return pl.pallas_call(
        body,
        grid=grid,
        in_specs=[
            pl.BlockSpec((TE, ED), lambda i: (i, 0)),
            pl.BlockSpec((TE, SD), lambda i: (i, 0)),
            pl.BlockSpec((ED, HID), lambda i: (0, 0)),
            pl.BlockSpec((1, HID), lambda i: (0, 0)),
            pl.BlockSpec((HID, SD * OC), lambda i: (0, 0)),
            pl.BlockSpec((1, SD * OC), lambda i: (0, 0)),
        ],
        out_specs=pl.BlockSpec((TE, OC), lambda i: (i, 0)),
        out_shape=jax.ShapeDtypeStruct((E, OC), jnp.float32),
    )(ea, xj, W1, b1.reshape(1, HID), W2, b2.reshape(1, SD * OC))


def _sc_scatter(msg, dst, z2d, z1d, ones_c):
    mesh = plsc.VectorSubcoreMesh(core_axis_name="c", subcore_axis_name="s")

    @functools.partial(
        pl.kernel,
        out_type=(jax.ShapeDtypeStruct((NC * NPAD, OC), jnp.float32),
                  jax.ShapeDtypeStruct((NC * NPAD,), jnp.float32)),
        mesh=mesh,
        scratch_types=[
            pltpu.VMEM((CH,), jnp.int32),
            pltpu.VMEM((CH, OC), jnp.float32),
            pltpu.VMEM((CH,), jnp.float32),
            pltpu.VMEM_SHARED((NPAD, OC), jnp.float32),
            pltpu.VMEM_SHARED((NPAD,), jnp.float32),
        ],
    )
    def scatter_k(msg_hbm, dst_hbm, z2d_hbm, z1d_hbm, ones_hbm,
                  agg_hbm, cnt_hbm, idx_v, msg_v, ones_v, acc_sh, cnt_sh):
        cid = lax.axis_index("c")
        sid = lax.axis_index("s")

        @pl.when(sid == 0)
        def _zero():
            pltpu.sync_copy(z2d_hbm, acc_sh)
            pltpu.sync_copy(z1d_hbm, cnt_sh)

        pltpu.sync_copy(ones_hbm, ones_v)
        plsc.subcore_barrier()

        base = (sid * NC + cid) * EPW

        def body(j, carry):
            off = base + j * CH
            pltpu.sync_copy(dst_hbm.at[pl.ds(off, CH)], idx_v)
            pltpu.sync_copy(msg_hbm.at[pl.ds(off, CH)], msg_v)
            pltpu.sync_copy(msg_v, acc_sh.at[idx_v], add=True)
            pltpu.sync_copy(ones_v, cnt_sh.at[idx_v], add=True)
            return carry

        lax.fori_loop(0, NCH, body, 0)
        plsc.subcore_barrier()

        ro = sid * RPT
        pltpu.sync_copy(acc_sh.at[pl.ds(ro, RPT)],
                        agg_hbm.at[pl.ds(cid * NPAD + ro, RPT)])
        pltpu.sync_copy(cnt_sh.at[pl.ds(ro, RPT)],
                        cnt_hbm.at[pl.ds(cid * NPAD + ro, RPT)])

    return scatter_k(msg, dst, z2d, z1d, ones_c)


def _tc_final(parts, cnts, xpad, maskf, act, root, bias,
              M1a, M1b, mb1, M2, mb2, M3, mb3):
    T = 1280
    G = NPAD // T

    def body(p0, p1, c0, c1, xr, mr, ar, root_r, bias_r,
             m1a, m1b, mb1_r, m2, mb2_r, m3, mb3_r, yr):
        cnt = jnp.maximum(c0[...] + c1[...], 1.0)
        agg = (p0[...] + p1[...]) / cnt
        out = agg + jnp.dot(xr[...], root_r[...],
                            preferred_element_type=jnp.float32) + bias_r[...]
        out = out * mr[...]
        v1 = (jnp.dot(out, m1a[...], preferred_element_type=jnp.float32)
              + ar[...] * m1b[...] + mb1_r[...])
        z1 = jnp.where(v1 > 0, v1, jnp.exp(jnp.minimum(v1, 0.0)) - 1.0)
        v2 = jnp.dot(z1, m2[...], preferred_element_type=jnp.float32) + mb2_r[...]
        z2 = jnp.where(v2 > 0, v2, jnp.exp(jnp.minimum(v2, 0.0)) - 1.0)
        yr[...] = jnp.dot(z2, m3[...], preferred_element_type=jnp.float32) + mb3_r[...]

    return pl.pallas_call(
        body,
        grid=(G,),
        in_specs=[
            pl.BlockSpec((T, OC), lambda i: (i, 0)),
            pl.BlockSpec((T, OC), lambda i: (i + G, 0)),
            pl.BlockSpec((T, 1), lambda i: (i, 0)),
            pl.BlockSpec((T, 1), lambda i: (i + G, 0)),
            pl.BlockSpec((T, SD), lambda i: (i, 0)),
            pl.BlockSpec((T, 1), lambda i: (i, 0)),
            pl.BlockSpec((T, 1), lambda i: (i, 0)),
            pl.BlockSpec((SD, OC), lambda i: (0, 0)),
            pl.BlockSpec((1, OC), lambda i: (0, 0)),
            pl.BlockSpec((OC, 64), lambda i: (0, 0)),
            pl.BlockSpec((1, 64), lambda i: (0, 0)),
            pl.BlockSpec((1, 64), lambda i: (0, 0)),
            pl.BlockSpec((64, 64), lambda i: (0, 0)),
            pl.BlockSpec((1, 64), lambda i: (0, 0)),
            pl.BlockSpec((64, 1), lambda i: (0, 0)),
            pl.BlockSpec((1, 1), lambda i: (0, 0)),
        ],
        out_specs=pl.BlockSpec((T, 1), lambda i: (i, 0)),
        out_shape=jax.ShapeDtypeStruct((NPAD, 1), jnp.float32),
    )(parts, parts, cnts, cnts, xpad, maskf, act, root, bias,
      M1a, M1b, mb1, M2, mb2, M3, mb3)


def kernel(x, edge_index, edge_attr, mask, batch, action,
           W1, b1, W2, b2, root, bias, M1, mb1, M2, mb2, M3, mb3):
    src = edge_index[0].astype(jnp.int32)
    dst = edge_index[1].astype(jnp.int32)

    xj = _sc_gather(x, src)
    msg = _tc_msg(edge_attr, xj, W1, b1, W2, b2)

    z2d = jnp.zeros((NPAD, OC), jnp.float32)
    z1d = jnp.zeros((NPAD,), jnp.float32)
    ones_c = jnp.ones((CH,), jnp.float32)
    parts, cnts = _sc_scatter(msg, dst, z2d, z1d, ones_c)

    pad = NPAD - N
    xpad = jnp.pad(x, ((0, pad), (0, 0)))
    maskf = jnp.pad(mask.astype(jnp.float32), (0, pad)).reshape(NPAD, 1)
    act = jnp.pad(action.astype(jnp.float32), (0, pad)).reshape(NPAD, 1)

    y = _tc_final(parts, cnts.reshape(NC * NPAD, 1), xpad, maskf, act,
                  root, bias.reshape(1, OC),
                  M1[:OC], M1[OC:OC + 1], mb1.reshape(1, 64),
                  M2, mb2.reshape(1, 64), M3, mb3.reshape(1, 1))
    return y[:N]


# speedup vs baseline: 1.5840x; 1.5840x over previous
import functools

import jax
import jax.numpy as jnp
from jax import lax
from jax.experimental import pallas as pl
from jax.experimental.pallas import tpu as pltpu
from jax.experimental.pallas import tpu_sc as plsc

N = 10000
E = 320000
SD = 16
OC = 16
ED = 16
HID = 16
NPAD = 10240

NC = 2
NS = 16
NW = NC * NS
EPW = E // NW
CH = 2000
NCH = EPW // CH
RPT = NPAD // NS


def _sc_gather(x, src):
    mesh = plsc.VectorSubcoreMesh(core_axis_name="c", subcore_axis_name="s")

    @functools.partial(
        pl.kernel,
        out_type=jax.ShapeDtypeStruct((E, SD), jnp.float32),
        mesh=mesh,
        scratch_types=[
            pltpu.VMEM((CH,), jnp.int32),
            pltpu.VMEM((CH, SD), jnp.float32),
            pltpu.SemaphoreType.DMA,
        ],
        compiler_params=pltpu.CompilerParams(use_tc_tiling_on_sc=False),
    )
    def gather_k(x_hbm, src_hbm, xj_hbm, idx_v, rows_v, sem):
        cid = lax.axis_index("c")
        sid = lax.axis_index("s")
        base = (sid * NC + cid) * EPW

        def body(j, carry):
            off = base + j * CH
            pltpu.sync_copy(src_hbm.at[pl.ds(off, CH)], idx_v)
            pltpu.async_copy(x_hbm.at[idx_v], rows_v, sem).wait()
            pltpu.sync_copy(rows_v, xj_hbm.at[pl.ds(off, CH)])
            return carry

        lax.fori_loop(0, NCH, body, 0)

    return gather_k(x, src)


def _tc_msg(ea, xj, W1, b1, W2, b2):
    TE = 2000
    grid = (E // TE,)

    def body(ea_ref, xj_ref, w1_ref, b1_ref, w2_ref, b2_ref, out_ref):
        h = jnp.maximum(
            jnp.dot(ea_ref[...], w1_ref[...], preferred_element_type=jnp.float32)
            + b1_ref[...], 0.0)
        w = jnp.dot(h, w2_ref[...], preferred_element_type=jnp.float32) + b2_ref[...]
        xjv = xj_ref[...]
        acc = xjv[:, 0:1] * w[:, 0:OC]
        for i in range(1, SD):
            acc = acc + xjv[:, i:i + 1] * w[:, i * OC:(i + 1) * OC]
        out_ref[...] = acc

    return pl.pallas_call(
        body,
        grid=grid,
        in_specs=[
            pl.BlockSpec((TE, ED), lambda i: (i, 0)),
            pl.BlockSpec((TE, SD), lambda i: (i, 0)),
            pl.BlockSpec((ED, HID), lambda i: (0, 0)),
            pl.BlockSpec((1, HID), lambda i: (0, 0)),
            pl.BlockSpec((HID, SD * OC), lambda i: (0, 0)),
            pl.BlockSpec((1, SD * OC), lambda i: (0, 0)),
        ],
        out_specs=pl.BlockSpec((TE, OC), lambda i: (i, 0)),
        out_shape=jax.ShapeDtypeStruct((E, OC), jnp.float32),
    )(ea, xj, W1, b1.reshape(1, HID), W2, b2.reshape(1, SD * OC))


def _sc_scatter(msg, dst, z2d, z1d, ones_c):
    mesh = plsc.VectorSubcoreMesh(core_axis_name="c", subcore_axis_name="s")

    @functools.partial(
        pl.kernel,
        out_type=(jax.ShapeDtypeStruct((NC * NPAD, OC), jnp.float32),
                  jax.ShapeDtypeStruct((NC * NPAD,), jnp.float32)),
        mesh=mesh,
        scratch_types=[
            pltpu.VMEM((CH,), jnp.int32),
            pltpu.VMEM((CH, OC), jnp.float32),
            pltpu.VMEM((CH,), jnp.float32),
            pltpu.VMEM_SHARED((NPAD, OC), jnp.float32),
            pltpu.VMEM_SHARED((NPAD,), jnp.float32),
        ],
        compiler_params=pltpu.CompilerParams(use_tc_tiling_on_sc=False),
    )
    def scatter_k(msg_hbm, dst_hbm, z2d_hbm, z1d_hbm, ones_hbm,
                  agg_hbm, cnt_hbm, idx_v, msg_v, ones_v, acc_sh, cnt_sh):
        cid = lax.axis_index("c")
        sid = lax.axis_index("s")

        @pl.when(sid == 0)
        def _zero():
            pltpu.sync_copy(z2d_hbm, acc_sh)
            pltpu.sync_copy(z1d_hbm, cnt_sh)

        pltpu.sync_copy(ones_hbm, ones_v)
        plsc.subcore_barrier()

        base = (sid * NC + cid) * EPW

        def body(j, carry):
            off = base + j * CH
            pltpu.sync_copy(dst_hbm.at[pl.ds(off, CH)], idx_v)
            pltpu.sync_copy(msg_hbm.at[pl.ds(off, CH)], msg_v)
            pltpu.sync_copy(msg_v, acc_sh.at[idx_v], add=True)
            pltpu.sync_copy(ones_v, cnt_sh.at[idx_v], add=True)
            return carry

        lax.fori_loop(0, NCH, body, 0)
        plsc.subcore_barrier()

        ro = sid * RPT
        pltpu.sync_copy(acc_sh.at[pl.ds(ro, RPT)],
                        agg_hbm.at[pl.ds(cid * NPAD + ro, RPT)])
        pltpu.sync_copy(cnt_sh.at[pl.ds(ro, RPT)],
                        cnt_hbm.at[pl.ds(cid * NPAD + ro, RPT)])

    return scatter_k(msg, dst, z2d, z1d, ones_c)


def _tc_final(parts, cnts, xpad, maskf, act, root, bias,
              M1a, M1b, mb1, M2, mb2, M3, mb3):
    T = 1280
    G = NPAD // T

    def body(p0, p1, c0, c1, xr, mr, ar, root_r, bias_r,
             m1a, m1b, mb1_r, m2, mb2_r, m3, mb3_r, yr):
        cnt = jnp.maximum(c0[...] + c1[...], 1.0)
        agg = (p0[...] + p1[...]) / cnt
        out = agg + jnp.dot(xr[...], root_r[...],
                            preferred_element_type=jnp.float32) + bias_r[...]
        out = out * mr[...]
        v1 = (jnp.dot(out, m1a[...], preferred_element_type=jnp.float32)
              + ar[...] * m1b[...] + mb1_r[...])
        z1 = jnp.where(v1 > 0, v1, jnp.exp(jnp.minimum(v1, 0.0)) - 1.0)
        v2 = jnp.dot(z1, m2[...], preferred_element_type=jnp.float32) + mb2_r[...]
        z2 = jnp.where(v2 > 0, v2, jnp.exp(jnp.minimum(v2, 0.0)) - 1.0)
        yr[...] = jnp.dot(z2, m3[...], preferred_element_type=jnp.float32) + mb3_r[...]

    return pl.pallas_call(
        body,
        grid=(G,),
        in_specs=[
            pl.BlockSpec((T, OC), lambda i: (i, 0)),
            pl.BlockSpec((T, OC), lambda i: (i + G, 0)),
            pl.BlockSpec((T, 1), lambda i: (i, 0)),
            pl.BlockSpec((T, 1), lambda i: (i + G, 0)),
            pl.BlockSpec((T, SD), lambda i: (i, 0)),
            pl.BlockSpec((T, 1), lambda i: (i, 0)),
            pl.BlockSpec((T, 1), lambda i: (i, 0)),
            pl.BlockSpec((SD, OC), lambda i: (0, 0)),
            pl.BlockSpec((1, OC), lambda i: (0, 0)),
            pl.BlockSpec((OC, 64), lambda i: (0, 0)),
            pl.BlockSpec((1, 64), lambda i: (0, 0)),
            pl.BlockSpec((1, 64), lambda i: (0, 0)),
            pl.BlockSpec((64, 64), lambda i: (0, 0)),
            pl.BlockSpec((1, 64), lambda i: (0, 0)),
            pl.BlockSpec((64, 1), lambda i: (0, 0)),
            pl.BlockSpec((1, 1), lambda i: (0, 0)),
        ],
        out_specs=pl.BlockSpec((T, 1), lambda i: (i, 0)),
        out_shape=jax.ShapeDtypeStruct((NPAD, 1), jnp.float32),
    )(parts, parts, cnts, cnts, xpad, maskf, act, root, bias,
      M1a, M1b, mb1, M2, mb2, M3, mb3)


def kernel(x, edge_index, edge_attr, mask, batch, action,
           W1, b1, W2, b2, root, bias, M1, mb1, M2, mb2, M3, mb3):
    src = edge_index[0].astype(jnp.int32)
    dst = edge_index[1].astype(jnp.int32)

    xj = _sc_gather(x, src)
    msg = _tc_msg(edge_attr, xj, W1, b1, W2, b2)

    z2d = jnp.zeros((NPAD, OC), jnp.float32)
    z1d = jnp.zeros((NPAD,), jnp.float32)
    ones_c = jnp.ones((CH,), jnp.float32)
    parts, cnts = _sc_scatter(msg, dst, z2d, z1d, ones_c)

    pad = NPAD - N
    xpad = jnp.pad(x, ((0, pad), (0, 0)))
    maskf = jnp.pad(mask.astype(jnp.float32), (0, pad)).reshape(NPAD, 1)
    act = jnp.pad(action.astype(jnp.float32), (0, pad)).reshape(NPAD, 1)

    y = _tc_final(parts, cnts.reshape(NC * NPAD, 1), xpad, maskf, act,
                  root, bias.reshape(1, OC),
                  M1[:OC], M1[OC:OC + 1], mb1.reshape(1, 64),
                  M2, mb2.reshape(1, 64), M3, mb3.reshape(1, 1))
    return y[:N]


# speedup vs baseline: 4.6427x; 2.9311x over previous
import functools

import jax
import jax.numpy as jnp
from jax import lax
from jax.experimental import pallas as pl
from jax.experimental.pallas import tpu as pltpu
from jax.experimental.pallas import tpu_sc as plsc

N = 10000
E = 320000
SD = 16
OC = 16
ED = 16
HID = 16
NPAD = 10240

NC = 2
NS = 16
NW = NC * NS
EPW = E // NW
CH = 2000
NCH = EPW // CH
RPT = NPAD // NS


def _sc_gather(x, src):
    mesh = plsc.VectorSubcoreMesh(core_axis_name="c", subcore_axis_name="s")

    @functools.partial(
        pl.kernel,
        out_type=jax.ShapeDtypeStruct((E, SD), jnp.float32),
        mesh=mesh,
        scratch_types=[
            pltpu.VMEM((CH,), jnp.int32),
            pltpu.VMEM((CH, SD), jnp.float32),
            pltpu.SemaphoreType.DMA,
        ],
        compiler_params=pltpu.CompilerParams(use_tc_tiling_on_sc=False),
    )
    def gather_k(x_hbm, src_hbm, xj_hbm, idx_v, rows_v, sem):
        cid = lax.axis_index("c")
        sid = lax.axis_index("s")
        base = (sid * NC + cid) * EPW

        def body(j, carry):
            off = base + j * CH
            pltpu.sync_copy(src_hbm.at[pl.ds(off, CH)], idx_v)
            pltpu.async_copy(x_hbm.at[idx_v], rows_v, sem).wait()
            pltpu.sync_copy(rows_v, xj_hbm.at[pl.ds(off, CH)])
            return carry

        lax.fori_loop(0, NCH, body, 0)

    return gather_k(x, src)


def _tc_msg(ea, xj, W1, b1, W2, b2):
    TE = 4000
    grid = (E // TE,)
    R = jnp.kron(jnp.eye(SD, dtype=jnp.float32), jnp.ones((1, OC), jnp.float32))
    S = jnp.kron(jnp.ones((SD, 1), jnp.float32), jnp.eye(OC, dtype=jnp.float32))

    def body(ea_ref, xj_ref, w1_ref, b1_ref, w2_ref, b2_ref, r_ref, s_ref,
             out_ref):
        h = jnp.maximum(
            jnp.dot(ea_ref[...], w1_ref[...], preferred_element_type=jnp.float32)
            + b1_ref[...], 0.0)
        w = jnp.dot(h, w2_ref[...], preferred_element_type=jnp.float32) + b2_ref[...]
        xr = jnp.dot(xj_ref[...], r_ref[...], preferred_element_type=jnp.float32)
        out_ref[...] = jnp.dot(xr * w, s_ref[...],
                               preferred_element_type=jnp.float32)

    return pl.pallas_call(
        body,
        grid=grid,
        in_specs=[
            pl.BlockSpec((TE, ED), lambda i: (i, 0)),
            pl.BlockSpec((TE, SD), lambda i: (i, 0)),
            pl.BlockSpec((ED, HID), lambda i: (0, 0)),
            pl.BlockSpec((1, HID), lambda i: (0, 0)),
            pl.BlockSpec((HID, SD * OC), lambda i: (0, 0)),
            pl.BlockSpec((1, SD * OC), lambda i: (0, 0)),
            pl.BlockSpec((SD, SD * OC), lambda i: (0, 0)),
            pl.BlockSpec((SD * OC, OC), lambda i: (0, 0)),
        ],
        out_specs=pl.BlockSpec((TE, OC), lambda i: (i, 0)),
        out_shape=jax.ShapeDtypeStruct((E, OC), jnp.float32),
    )(ea, xj, W1, b1.reshape(1, HID), W2, b2.reshape(1, SD * OC), R, S)


def _sc_scatter(msg, dst, z2d, z1d, ones_c):
    mesh = plsc.VectorSubcoreMesh(core_axis_name="c", subcore_axis_name="s")

    @functools.partial(
        pl.kernel,
        out_type=(jax.ShapeDtypeStruct((NC * NPAD, OC), jnp.float32),
                  jax.ShapeDtypeStruct((NC * NPAD,), jnp.float32)),
        mesh=mesh,
        scratch_types=[
            pltpu.VMEM((CH,), jnp.int32),
            pltpu.VMEM((CH, OC), jnp.float32),
            pltpu.VMEM((CH,), jnp.float32),
            pltpu.VMEM_SHARED((NPAD, OC), jnp.float32),
            pltpu.VMEM_SHARED((NPAD,), jnp.float32),
        ],
        compiler_params=pltpu.CompilerParams(use_tc_tiling_on_sc=False),
    )
    def scatter_k(msg_hbm, dst_hbm, z2d_hbm, z1d_hbm, ones_hbm,
                  agg_hbm, cnt_hbm, idx_v, msg_v, ones_v, acc_sh, cnt_sh):
        cid = lax.axis_index("c")
        sid = lax.axis_index("s")

        @pl.when(sid == 0)
        def _zero():
            pltpu.sync_copy(z2d_hbm, acc_sh)
            pltpu.sync_copy(z1d_hbm, cnt_sh)

        pltpu.sync_copy(ones_hbm, ones_v)
        plsc.subcore_barrier()

        base = (sid * NC + cid) * EPW

        def body(j, carry):
            off = base + j * CH
            pltpu.sync_copy(dst_hbm.at[pl.ds(off, CH)], idx_v)
            pltpu.sync_copy(msg_hbm.at[pl.ds(off, CH)], msg_v)
            pltpu.sync_copy(msg_v, acc_sh.at[idx_v], add=True)
            pltpu.sync_copy(ones_v, cnt_sh.at[idx_v], add=True)
            return carry

        lax.fori_loop(0, NCH, body, 0)
        plsc.subcore_barrier()

        ro = sid * RPT
        pltpu.sync_copy(acc_sh.at[pl.ds(ro, RPT)],
                        agg_hbm.at[pl.ds(cid * NPAD + ro, RPT)])
        pltpu.sync_copy(cnt_sh.at[pl.ds(ro, RPT)],
                        cnt_hbm.at[pl.ds(cid * NPAD + ro, RPT)])

    return scatter_k(msg, dst, z2d, z1d, ones_c)


def _tc_final(parts, cnts, xpad, maskf, act, root, bias,
              M1a, M1b, mb1, M2, mb2, M3, mb3):
    T = 1280
    G = NPAD // T

    def body(p0, p1, c0, c1, xr, mr, ar, root_r, bias_r,
             m1a, m1b, mb1_r, m2, mb2_r, m3, mb3_r, yr):
        cnt = jnp.maximum(c0[...] + c1[...], 1.0)
        agg = (p0[...] + p1[...]) / cnt
        out = agg + jnp.dot(xr[...], root_r[...],
                            preferred_element_type=jnp.float32) + bias_r[...]
        out = out * mr[...]
        v1 = (jnp.dot(out, m1a[...], preferred_element_type=jnp.float32)
              + ar[...] * m1b[...] + mb1_r[...])
        z1 = jnp.where(v1 > 0, v1, jnp.exp(jnp.minimum(v1, 0.0)) - 1.0)
        v2 = jnp.dot(z1, m2[...], preferred_element_type=jnp.float32) + mb2_r[...]
        z2 = jnp.where(v2 > 0, v2, jnp.exp(jnp.minimum(v2, 0.0)) - 1.0)
        yr[...] = jnp.dot(z2, m3[...], preferred_element_type=jnp.float32) + mb3_r[...]

    return pl.pallas_call(
        body,
        grid=(G,),
        in_specs=[
            pl.BlockSpec((T, OC), lambda i: (i, 0)),
            pl.BlockSpec((T, OC), lambda i: (i + G, 0)),
            pl.BlockSpec((T, 1), lambda i: (i, 0)),
            pl.BlockSpec((T, 1), lambda i: (i + G, 0)),
            pl.BlockSpec((T, SD), lambda i: (i, 0)),
            pl.BlockSpec((T, 1), lambda i: (i, 0)),
            pl.BlockSpec((T, 1), lambda i: (i, 0)),
            pl.BlockSpec((SD, OC), lambda i: (0, 0)),
            pl.BlockSpec((1, OC), lambda i: (0, 0)),
            pl.BlockSpec((OC, 64), lambda i: (0, 0)),
            pl.BlockSpec((1, 64), lambda i: (0, 0)),
            pl.BlockSpec((1, 64), lambda i: (0, 0)),
            pl.BlockSpec((64, 64), lambda i: (0, 0)),
            pl.BlockSpec((1, 64), lambda i: (0, 0)),
            pl.BlockSpec((64, 1), lambda i: (0, 0)),
            pl.BlockSpec((1, 1), lambda i: (0, 0)),
        ],
        out_specs=pl.BlockSpec((T, 1), lambda i: (i, 0)),
        out_shape=jax.ShapeDtypeStruct((NPAD, 1), jnp.float32),
    )(parts, parts, cnts, cnts, xpad, maskf, act, root, bias,
      M1a, M1b, mb1, M2, mb2, M3, mb3)


def kernel(x, edge_index, edge_attr, mask, batch, action,
           W1, b1, W2, b2, root, bias, M1, mb1, M2, mb2, M3, mb3):
    src = edge_index[0].astype(jnp.int32)
    dst = edge_index[1].astype(jnp.int32)

    xj = _sc_gather(x, src)
    msg = _tc_msg(edge_attr, xj, W1, b1, W2, b2)

    z2d = jnp.zeros((NPAD, OC), jnp.float32)
    z1d = jnp.zeros((NPAD,), jnp.float32)
    ones_c = jnp.ones((CH,), jnp.float32)
    parts, cnts = _sc_scatter(msg, dst, z2d, z1d, ones_c)

    pad = NPAD - N
    xpad = jnp.pad(x, ((0, pad), (0, 0)))
    maskf = jnp.pad(mask.astype(jnp.float32), (0, pad)).reshape(NPAD, 1)
    act = jnp.pad(action.astype(jnp.float32), (0, pad)).reshape(NPAD, 1)

    y = _tc_final(parts, cnts.reshape(NC * NPAD, 1), xpad, maskf, act,
                  root, bias.reshape(1, OC),
                  M1[:OC], M1[OC:OC + 1], mb1.reshape(1, 64),
                  M2, mb2.reshape(1, 64), M3, mb3.reshape(1, 1))
    return y[:N]


# speedup vs baseline: 6.8514x; 1.4757x over previous
import functools

import jax
import jax.numpy as jnp
from jax import lax
from jax.experimental import pallas as pl
from jax.experimental.pallas import tpu as pltpu
from jax.experimental.pallas import tpu_sc as plsc

N = 10000
E = 320000
SD = 16
OC = 16
ED = 16
HID = 16
NPAD = 10240

NC = 2
NS = 16
NW = NC * NS
EPW = E // NW
CH = 2000
NCH = EPW // CH
RPT = NPAD // NS


def _sc_gather(x, src):
    mesh = plsc.VectorSubcoreMesh(core_axis_name="c", subcore_axis_name="s")

    @functools.partial(
        pl.kernel,
        out_type=jax.ShapeDtypeStruct((E, SD), jnp.float32),
        mesh=mesh,
        scratch_types=[
            pltpu.VMEM((CH,), jnp.int32),
            pltpu.VMEM((CH, SD), jnp.float32),
            pltpu.SemaphoreType.DMA,
        ],
        compiler_params=pltpu.CompilerParams(use_tc_tiling_on_sc=False),
    )
    def gather_k(x_hbm, src_hbm, xj_hbm, idx_v, rows_v, sem):
        cid = lax.axis_index("c")
        sid = lax.axis_index("s")
        base = (sid * NC + cid) * EPW

        def body(j, carry):
            off = base + j * CH
            pltpu.sync_copy(src_hbm.at[pl.ds(off, CH)], idx_v)
            pltpu.async_copy(x_hbm.at[idx_v], rows_v, sem).wait()
            pltpu.sync_copy(rows_v, xj_hbm.at[pl.ds(off, CH)])
            return carry

        lax.fori_loop(0, NCH, body, 0)

    return gather_k(x, src)


def _tc_msg(ea_pk, xj_pk, W1, b1, W2, b2):
    E8 = E // 8
    TB = 1000
    grid = (E8 // TB,)
    ey8 = jnp.eye(8, dtype=jnp.float32)
    R = jnp.kron(jnp.eye(SD, dtype=jnp.float32), jnp.ones((1, OC), jnp.float32))
    S = jnp.kron(jnp.ones((SD, 1), jnp.float32), jnp.eye(OC, dtype=jnp.float32))
    W1b = jnp.kron(ey8, W1)
    b1b = jnp.tile(b1, 8).reshape(1, 128)
    W2b = jnp.kron(ey8, W2)
    b2b = jnp.tile(b2, 8).reshape(1, 8 * SD * OC)
    Rb = jnp.kron(ey8, R)
    Sb = jnp.kron(ey8, S)

    def body(ea_ref, xj_ref, w1_ref, b1_ref, w2_ref, b2_ref, r_ref, s_ref,
             out_ref):
        h = jnp.maximum(
            jnp.dot(ea_ref[...], w1_ref[...], preferred_element_type=jnp.float32)
            + b1_ref[...], 0.0)
        w = jnp.dot(h, w2_ref[...], preferred_element_type=jnp.float32) + b2_ref[...]
        xr = jnp.dot(xj_ref[...], r_ref[...], preferred_element_type=jnp.float32)
        out_ref[...] = jnp.dot(xr * w, s_ref[...],
                               preferred_element_type=jnp.float32)

    C = 8 * SD * OC
    return pl.pallas_call(
        body,
        grid=grid,
        in_specs=[
            pl.BlockSpec((TB, 128), lambda i: (i, 0)),
            pl.BlockSpec((TB, 128), lambda i: (i, 0)),
            pl.BlockSpec((128, 128), lambda i: (0, 0)),
            pl.BlockSpec((1, 128), lambda i: (0, 0)),
            pl.BlockSpec((128, C), lambda i: (0, 0)),
            pl.BlockSpec((1, C), lambda i: (0, 0)),
            pl.BlockSpec((128, C), lambda i: (0, 0)),
            pl.BlockSpec((C, 128), lambda i: (0, 0)),
        ],
        out_specs=pl.BlockSpec((TB, 128), lambda i: (i, 0)),
        out_shape=jax.ShapeDtypeStruct((E8, 128), jnp.float32),
    )(ea_pk, xj_pk, W1b, b1b, W2b, b2b, Rb, Sb)


def _sc_scatter(msg, dst, z2d, z1d, ones_c):
    mesh = plsc.VectorSubcoreMesh(core_axis_name="c", subcore_axis_name="s")

    @functools.partial(
        pl.kernel,
        out_type=(jax.ShapeDtypeStruct((NC * NPAD, OC), jnp.float32),
                  jax.ShapeDtypeStruct((NC * NPAD,), jnp.float32)),
        mesh=mesh,
        scratch_types=[
            pltpu.VMEM((CH,), jnp.int32),
            pltpu.VMEM((CH, OC), jnp.float32),
            pltpu.VMEM((CH,), jnp.float32),
            pltpu.VMEM_SHARED((NPAD, OC), jnp.float32),
            pltpu.VMEM_SHARED((NPAD,), jnp.float32),
        ],
        compiler_params=pltpu.CompilerParams(use_tc_tiling_on_sc=False),
    )
    def scatter_k(msg_hbm, dst_hbm, z2d_hbm, z1d_hbm, ones_hbm,
                  agg_hbm, cnt_hbm, idx_v, msg_v, ones_v, acc_sh, cnt_sh):
        cid = lax.axis_index("c")
        sid = lax.axis_index("s")

        @pl.when(sid == 0)
        def _zero():
            pltpu.sync_copy(z2d_hbm, acc_sh)
            pltpu.sync_copy(z1d_hbm, cnt_sh)

        pltpu.sync_copy(ones_hbm, ones_v)
        plsc.subcore_barrier()

        base = (sid * NC + cid) * EPW

        def body(j, carry):
            off = base + j * CH
            pltpu.sync_copy(dst_hbm.at[pl.ds(off, CH)], idx_v)
            pltpu.sync_copy(msg_hbm.at[pl.ds(off, CH)], msg_v)
            pltpu.sync_copy(msg_v, acc_sh.at[idx_v], add=True)
            pltpu.sync_copy(ones_v, cnt_sh.at[idx_v], add=True)
            return carry

        lax.fori_loop(0, NCH, body, 0)
        plsc.subcore_barrier()

        ro = sid * RPT
        pltpu.sync_copy(acc_sh.at[pl.ds(ro, RPT)],
                        agg_hbm.at[pl.ds(cid * NPAD + ro, RPT)])
        pltpu.sync_copy(cnt_sh.at[pl.ds(ro, RPT)],
                        cnt_hbm.at[pl.ds(cid * NPAD + ro, RPT)])

    return scatter_k(msg, dst, z2d, z1d, ones_c)


def _tc_final(parts, cnts, xpad, maskf, act, root, bias,
              M1a, M1b, mb1, M2, mb2, M3, mb3):
    T = 1280
    G = NPAD // T

    def body(p0, p1, c0, c1, xr, mr, ar, root_r, bias_r,
             m1a, m1b, mb1_r, m2, mb2_r, m3, mb3_r, yr):
        cnt = jnp.maximum(c0[...] + c1[...], 1.0)
        agg = (p0[...] + p1[...]) / cnt
        out = agg + jnp.dot(xr[...], root_r[...],
                            preferred_element_type=jnp.float32) + bias_r[...]
        out = out * mr[...]
        v1 = (jnp.dot(out, m1a[...], preferred_element_type=jnp.float32)
              + ar[...] * m1b[...] + mb1_r[...])
        z1 = jnp.where(v1 > 0, v1, jnp.exp(jnp.minimum(v1, 0.0)) - 1.0)
        v2 = jnp.dot(z1, m2[...], preferred_element_type=jnp.float32) + mb2_r[...]
        z2 = jnp.where(v2 > 0, v2, jnp.exp(jnp.minimum(v2, 0.0)) - 1.0)
        yr[...] = jnp.dot(z2, m3[...], preferred_element_type=jnp.float32) + mb3_r[...]

    return pl.pallas_call(
        body,
        grid=(G,),
        in_specs=[
            pl.BlockSpec((T, OC), lambda i: (i, 0)),
            pl.BlockSpec((T, OC), lambda i: (i + G, 0)),
            pl.BlockSpec((T, 1), lambda i: (i, 0)),
            pl.BlockSpec((T, 1), lambda i: (i + G, 0)),
            pl.BlockSpec((T, SD), lambda i: (i, 0)),
            pl.BlockSpec((T, 1), lambda i: (i, 0)),
            pl.BlockSpec((T, 1), lambda i: (i, 0)),
            pl.BlockSpec((SD, OC), lambda i: (0, 0)),
            pl.BlockSpec((1, OC), lambda i: (0, 0)),
            pl.BlockSpec((OC, 64), lambda i: (0, 0)),
            pl.BlockSpec((1, 64), lambda i: (0, 0)),
            pl.BlockSpec((1, 64), lambda i: (0, 0)),
            pl.BlockSpec((64, 64), lambda i: (0, 0)),
            pl.BlockSpec((1, 64), lambda i: (0, 0)),
            pl.BlockSpec((64, 1), lambda i: (0, 0)),
            pl.BlockSpec((1, 1), lambda i: (0, 0)),
        ],
        out_specs=pl.BlockSpec((T, 1), lambda i: (i, 0)),
        out_shape=jax.ShapeDtypeStruct((NPAD, 1), jnp.float32),
    )(parts, parts, cnts, cnts, xpad, maskf, act, root, bias,
      M1a, M1b, mb1, M2, mb2, M3, mb3)


def kernel(x, edge_index, edge_attr, mask, batch, action,
           W1, b1, W2, b2, root, bias, M1, mb1, M2, mb2, M3, mb3):
    src = edge_index[0].astype(jnp.int32)
    dst = edge_index[1].astype(jnp.int32)

    xj = _sc_gather(x, src)
    ea_pk = edge_attr.reshape(E // 8, 8 * ED)
    xj_pk = xj.reshape(E // 8, 8 * SD)
    msg_pk = _tc_msg(ea_pk, xj_pk, W1, b1, W2, b2)
    msg = msg_pk.reshape(E, OC)

    z2d = jnp.zeros((NPAD, OC), jnp.float32)
    z1d = jnp.zeros((NPAD,), jnp.float32)
    ones_c = jnp.ones((CH,), jnp.float32)
    parts, cnts = _sc_scatter(msg, dst, z2d, z1d, ones_c)

    pad = NPAD - N
    xpad = jnp.pad(x, ((0, pad), (0, 0)))
    maskf = jnp.pad(mask.astype(jnp.float32), (0, pad)).reshape(NPAD, 1)
    act = jnp.pad(action.astype(jnp.float32), (0, pad)).reshape(NPAD, 1)

    y = _tc_final(parts, cnts.reshape(NC * NPAD, 1), xpad, maskf, act,
                  root, bias.reshape(1, OC),
                  M1[:OC], M1[OC:OC + 1], mb1.reshape(1, 64),
                  M2, mb2.reshape(1, 64), M3, mb3.reshape(1, 1))
    return y[:N]
